# baseline (speedup 1.0000x reference)
import jax
import jax.numpy as jnp
from jax.experimental import pallas as pl
from jax.experimental.pallas import tpu as pltpu


def _dma_copy(x_ref, o_ref, sem):
    pltpu.make_async_copy(x_ref, o_ref, sem).start()
    pltpu.make_async_copy(x_ref, o_ref, sem).wait()


def kernel(inputs_embeds):
    return pl.pallas_call(
        _dma_copy,
        in_specs=[pl.BlockSpec(memory_space=pltpu.MemorySpace.HBM)],
        out_specs=pl.BlockSpec(memory_space=pltpu.MemorySpace.HBM),
        out_shape=jax.ShapeDtypeStruct(inputs_embeds.shape, inputs_embeds.dtype),
        scratch_shapes=[pltpu.SemaphoreType.DMA],
    )(inputs_embeds)


# speedup vs baseline: 15.7118x; 15.7118x over previous
import jax
import jax.numpy as jnp
from jax.experimental import pallas as pl
from jax.experimental.pallas import tpu as pltpu

_NCHUNKS = 8


def _copy_kernel(x_ref, o_ref, scratch, *sems):
    in_sems = sems[:_NCHUNKS]
    out_sems = sems[_NCHUNKS:]
    seq = x_ref.shape[1]
    chunk = seq // _NCHUNKS

    def in_copy(i):
        sl = pl.ds(i * chunk, chunk)
        return pltpu.make_async_copy(
            x_ref.at[:, sl], scratch.at[:, sl], in_sems[i]
        )

    def out_copy(i):
        sl = pl.ds(i * chunk, chunk)
        return pltpu.make_async_copy(
            scratch.at[:, sl], o_ref.at[:, sl], out_sems[i]
        )

    for i in range(_NCHUNKS):
        in_copy(i).start()
    for i in range(_NCHUNKS):
        in_copy(i).wait()
        out_copy(i).start()
    for i in range(_NCHUNKS):
        out_copy(i).wait()


def kernel(inputs_embeds):
    shape = inputs_embeds.shape
    return pl.pallas_call(
        _copy_kernel,
        in_specs=[pl.BlockSpec(memory_space=pltpu.MemorySpace.HBM)],
        out_specs=pl.BlockSpec(memory_space=pltpu.MemorySpace.HBM),
        out_shape=jax.ShapeDtypeStruct(shape, inputs_embeds.dtype),
        scratch_shapes=(
            [pltpu.VMEM(shape, inputs_embeds.dtype)]
            + [pltpu.SemaphoreType.DMA] * (2 * _NCHUNKS)
        ),
    )(inputs_embeds)
